# baseline (speedup 1.0000x reference)
import functools

import jax
import jax.numpy as jnp
from jax import lax
from jax.experimental import pallas as pl
from jax.experimental.pallas import tpu as pltpu
from jax.experimental.pallas import tpu_sc as plsc

N = 50000
D = 64
E = 800000

NC = 2
NS = 16
HALF = N // NC
ACC_ROWS = 25600
ROWS_PER_TILE = ACC_ROWS // NS
TRASH_MASK = 511

KC = 128
NCHUNK = E // KC
CHT = NCHUNK // NS
LEFT = CHT * NS
GS = 16
SUP = CHT // GS
TAILC = CHT - SUP * GS

_mesh = plsc.VectorSubcoreMesh(
    core_axis_name="c", subcore_axis_name="s", num_cores=NC, num_subcores=NS)


DW = 16
NPT = N // NS


@functools.partial(
    pl.kernel,
    out_type=jax.ShapeDtypeStruct((NC, N, DW), jnp.float32),
    mesh=_mesh,
    scratch_types=[
        pltpu.VMEM((CHT, KC), jnp.int32),
        pltpu.VMEM((KC, DW), jnp.float32),
        pltpu.SemaphoreType.DMA,
        pltpu.VMEM_SHARED((N, DW), jnp.float32),
    ],
    compiler_params=pltpu.CompilerParams(use_tc_tiling_on_sc=False),
)
def _deg_kernel(edges3, ones_hbm, zeros_hbm, deg_out, idx2, onesv, sem, deg_sp):
    c = lax.axis_index("c")
    s = lax.axis_index("s")
    pltpu.sync_copy(ones_hbm, onesv)
    pltpu.sync_copy(zeros_hbm, deg_sp.at[pl.ds(s * NPT, NPT)])
    pltpu.sync_copy(edges3.at[c, pl.ds(s * CHT, CHT)], idx2)
    plsc.subcore_barrier()

    WIN = 6

    def fire(j, carry):
        pltpu.async_copy(onesv, deg_sp.at[idx2.at[j]], sem, add=True)

        @pl.when(j >= WIN)
        def _():
            pltpu.make_async_copy(onesv, deg_sp.at[idx2.at[j]], sem).wait()
        return carry
    lax.fori_loop(0, CHT, fire, 0)

    def drain(j, carry):
        pltpu.make_async_copy(onesv, deg_sp.at[idx2.at[0]], sem).wait()
        return carry
    lax.fori_loop(0, WIN, drain, 0)

    @pl.when(s < NCHUNK - LEFT)
    def _extra():
        pltpu.sync_copy(edges3.at[c, pl.ds(LEFT + s, 1)], idx2.at[pl.ds(0, 1)])
        pltpu.sync_copy(onesv, deg_sp.at[idx2.at[0]], add=True)

    plsc.subcore_barrier()
    pltpu.sync_copy(deg_sp.at[pl.ds(s * NPT, NPT)],
                    deg_out.at[c, pl.ds(s * NPT, NPT)])


@functools.partial(
    pl.kernel,
    out_type=jax.ShapeDtypeStruct((NC, ACC_ROWS, D), jnp.float32),
    mesh=_mesh,
    scratch_types=[
        pltpu.VMEM((GS, KC), jnp.int32),
        pltpu.VMEM((GS, KC), jnp.int32),
        pltpu.VMEM((GS * KC + 16,), jnp.int32),
        pltpu.VMEM((GS * KC + 16,), jnp.int32),
        pltpu.VMEM((KC, D), jnp.float32),
        pltpu.VMEM((KC, D), jnp.float32),
        pltpu.SemaphoreType.DMA,
        pltpu.SemaphoreType.DMA,
        pltpu.SemaphoreType.DMA,
        pltpu.SemaphoreType.DMA,
        pltpu.VMEM_SHARED((ACC_ROWS, D), jnp.float32),
    ],
    compiler_params=pltpu.CompilerParams(
        use_tc_tiling_on_sc=False, needs_layout_passes=False),
)
def _agg_kernel(x, edges3, zrows, out, src2, dst2, csrc, clidx,
                rowsA, rowsB, gA, gB, tA, tB, acc):
    c = lax.axis_index("c")
    s = lax.axis_index("s")
    lo = c * HALF

    pltpu.sync_copy(zrows, acc.at[pl.ds(s * ROWS_PER_TILE, ROWS_PER_TILE)])
    plsc.subcore_barrier()

    def lidx_row(j):
        for i in range(KC // 16):
            d = dst2[j, pl.ds(i * 16, 16)]
            inb = (d >= lo) & (d < lo + HALF)
            dst2[j, pl.ds(i * 16, 16)] = jnp.where(
                inb, d - lo, HALF + (d & TRASH_MASK))

    def start_gather(j, buf, sem):
        pltpu.async_copy(x.at[csrc.at[pl.ds(j * KC, KC)]], buf, sem)

    def wait_gather(buf, sem):
        pltpu.make_async_copy(x.at[csrc.at[pl.ds(0, KC)]], buf, sem).wait()

    def start_scatter(j, buf, sem):
        pltpu.async_copy(buf, acc.at[clidx.at[pl.ds(j * KC, KC)]], sem,
                         add=True)

    def wait_scatter(buf, sem):
        pltpu.make_async_copy(buf, acc.at[clidx.at[pl.ds(0, KC)]], sem).wait()

    def super_body(base, nmicro):
        pltpu.sync_copy(edges3.at[0, pl.ds(base, nmicro)],
                        src2.at[pl.ds(0, nmicro)])
        pltpu.sync_copy(edges3.at[1, pl.ds(base, nmicro)],
                        dst2.at[pl.ds(0, nmicro)])

        def comp(v, off):
            r = v >> 3
            cb = (v & 7) * 16
            d = dst2[r, pl.ds(cb, 16)]
            sv = src2[r, pl.ds(cb, 16)]
            inb = (d >= lo) & (d < lo + HALF)
            mi = inb.astype(jnp.int32)
            cum = plsc.cumsum(mi)
            pos = jnp.where(inb, off + cum - mi,
                            GS * KC + lax.iota(jnp.int32, 16))
            plsc.store_scatter(clidx, [pos], d - lo)
            plsc.store_scatter(csrc, [pos], sv)
            pc = plsc.all_reduce_population_count(inb)
            return off + pc[0]
        off = lax.fori_loop(0, nmicro * (KC // 16), comp, 0)

        nmic = (off + KC - 1) >> 7

        def pad(t, carry):
            pos = t * 16 + lax.iota(jnp.int32, 16)
            m = pos >= off
            cl = clidx[pl.ds(t * 16, 16)]
            cs = csrc[pl.ds(t * 16, 16)]
            clidx[pl.ds(t * 16, 16)] = jnp.where(
                m, HALF + (pos & TRASH_MASK), cl)
            csrc[pl.ds(t * 16, 16)] = jnp.where(m, pos, cs)
            return carry
        lax.fori_loop(off >> 4, (nmic * KC) >> 4, pad, 0)

        def fire(m, carry):
            j0 = 2 * m
            j1 = j0 + 1

            @pl.when(m > 0)
            def _():
                wait_scatter(rowsA, tA)
            start_gather(j0, rowsA, gA)

            @pl.when(m > 0)
            def _():
                wait_scatter(rowsB, tB)

            @pl.when(j1 < nmic)
            def _():
                start_gather(j1, rowsB, gB)
            wait_gather(rowsA, gA)
            start_scatter(j0, rowsA, tA)

            @pl.when(j1 < nmic)
            def _():
                wait_gather(rowsB, gB)
                start_scatter(j1, rowsB, tB)
            return carry
        lax.fori_loop(0, (nmic + 1) >> 1, fire, 0)

        @pl.when(nmic > 0)
        def _():
            wait_scatter(rowsA, tA)

        @pl.when((nmic > 0) & ((nmic & 1) == 0))
        def _():
            wait_scatter(rowsB, tB)

    def outer(u, carry):
        super_body(s * CHT + u * GS, GS)
        return carry
    lax.fori_loop(0, SUP, outer, 0)
    if TAILC:
        super_body(s * CHT + SUP * GS, TAILC)

    @pl.when(s < NCHUNK - LEFT)
    def _extra():
        pltpu.sync_copy(edges3.at[0, pl.ds(LEFT + s, 1)], src2.at[pl.ds(0, 1)])
        pltpu.sync_copy(edges3.at[1, pl.ds(LEFT + s, 1)], dst2.at[pl.ds(0, 1)])
        lidx_row(0)
        pltpu.sync_copy(x.at[src2.at[0]], rowsA)
        pltpu.sync_copy(rowsA, acc.at[dst2.at[0]], add=True)

    plsc.subcore_barrier()
    pltpu.sync_copy(acc.at[pl.ds(s * ROWS_PER_TILE, ROWS_PER_TILE)],
                    out.at[c, pl.ds(s * ROWS_PER_TILE, ROWS_PER_TILE)])


BM = 2048


def _mm_body(embT_ref, deg_ref, w_ref, x_ref):
    ns = lax.rsqrt(jnp.clip(deg_ref[0, :, 0:1], 1.0, None))
    y = lax.dot_general(embT_ref[...], w_ref[...], (((0,), (0,)), ((), ())),
                        precision=lax.Precision.HIGHEST,
                        preferred_element_type=jnp.float32)
    x_ref[...] = y * ns


def _mm_call(embT, deg, W):
    return pl.pallas_call(
        _mm_body,
        grid=(-(-N // BM),),
        in_specs=[
            pl.BlockSpec((D, BM), lambda i: (0, i)),
            pl.BlockSpec((1, BM, DW), lambda i: (0, i, 0)),
            pl.BlockSpec((D, D), lambda i: (0, 0)),
        ],
        out_specs=pl.BlockSpec((BM, D), lambda i: (i, 0)),
        out_shape=jax.ShapeDtypeStruct((N, D), jnp.float32),
    )(embT, deg, W)


BD = 1000
_HB = HALF // BD


def _fin_body(agg_ref, deg_ref, b_ref, out_ref):
    nd = lax.rsqrt(jnp.clip(deg_ref[0, :, 0:1], 1.0, None))
    out_ref[...] = agg_ref[0] * nd + b_ref[...]


def _fin_call(aggs, deg, b2d):
    return pl.pallas_call(
        _fin_body,
        grid=(N // BD,),
        in_specs=[
            pl.BlockSpec((1, BD, D), lambda i: (i // _HB, i % _HB, 0)),
            pl.BlockSpec((1, BD, DW), lambda i: (1, i, 0)),
            pl.BlockSpec((1, D), lambda i: (0, 0)),
        ],
        out_specs=pl.BlockSpec((BD, D), lambda i: (i, 0)),
        out_shape=jax.ShapeDtypeStruct((N, D), jnp.float32),
    )(aggs, deg, b2d)


def kernel(node_embeddings, W, b, edge_index):
    edges3 = edge_index.astype(jnp.int32).reshape(2, NCHUNK, KC)
    ones16 = jnp.ones((KC, DW), jnp.float32)
    zeros16 = jnp.zeros((NPT, DW), jnp.float32)
    deg = _deg_kernel(edges3, ones16, zeros16)
    x = _mm_call(node_embeddings.T, deg, W)
    zrows = jnp.zeros((ROWS_PER_TILE, D), jnp.float32)
    aggs = _agg_kernel(x, edges3, zrows)
    return _fin_call(aggs, deg, b.reshape(1, D))


# speedup vs baseline: 1.0091x; 1.0091x over previous
import functools

import jax
import jax.numpy as jnp
from jax import lax
from jax.experimental import pallas as pl
from jax.experimental.pallas import tpu as pltpu
from jax.experimental.pallas import tpu_sc as plsc

N = 50000
D = 64
E = 800000

NC = 2
NS = 16
HALF = N // NC
ACC_ROWS = 25600
ROWS_PER_TILE = ACC_ROWS // NS
TRASH_MASK = 511

KC = 128
NCHUNK = E // KC
CHT = NCHUNK // NS
LEFT = CHT * NS
GS = 16
SUP = CHT // GS
TAILC = CHT - SUP * GS

_mesh = plsc.VectorSubcoreMesh(
    core_axis_name="c", subcore_axis_name="s", num_cores=NC, num_subcores=NS)


DW = 16
NPT = N // NS


@functools.partial(
    pl.kernel,
    out_type=jax.ShapeDtypeStruct((NC, N, DW), jnp.float32),
    mesh=_mesh,
    scratch_types=[
        pltpu.VMEM((CHT, KC), jnp.int32),
        pltpu.VMEM((KC, DW), jnp.float32),
        pltpu.SemaphoreType.DMA,
        pltpu.VMEM_SHARED((N, DW), jnp.float32),
    ],
    compiler_params=pltpu.CompilerParams(use_tc_tiling_on_sc=False),
)
def _deg_kernel(edges3, ones_hbm, zeros_hbm, deg_out, idx2, onesv, sem, deg_sp):
    c = lax.axis_index("c")
    s = lax.axis_index("s")
    pltpu.sync_copy(ones_hbm, onesv)
    pltpu.sync_copy(zeros_hbm, deg_sp.at[pl.ds(s * NPT, NPT)])
    pltpu.sync_copy(edges3.at[c, pl.ds(s * CHT, CHT)], idx2)
    plsc.subcore_barrier()

    WIN = 6

    def fire(j, carry):
        pltpu.async_copy(onesv, deg_sp.at[idx2.at[j]], sem, add=True)

        @pl.when(j >= WIN)
        def _():
            pltpu.make_async_copy(onesv, deg_sp.at[idx2.at[j]], sem).wait()
        return carry
    lax.fori_loop(0, CHT, fire, 0)

    def drain(j, carry):
        pltpu.make_async_copy(onesv, deg_sp.at[idx2.at[0]], sem).wait()
        return carry
    lax.fori_loop(0, WIN, drain, 0)

    @pl.when(s < NCHUNK - LEFT)
    def _extra():
        pltpu.sync_copy(edges3.at[c, pl.ds(LEFT + s, 1)], idx2.at[pl.ds(0, 1)])
        pltpu.sync_copy(onesv, deg_sp.at[idx2.at[0]], add=True)

    plsc.subcore_barrier()
    pltpu.sync_copy(deg_sp.at[pl.ds(s * NPT, NPT)],
                    deg_out.at[c, pl.ds(s * NPT, NPT)])


@functools.partial(
    pl.kernel,
    out_type=jax.ShapeDtypeStruct((NC, ACC_ROWS, D), jnp.float32),
    mesh=_mesh,
    scratch_types=[
        pltpu.VMEM((GS, KC), jnp.int32),
        pltpu.VMEM((GS, KC), jnp.int32),
        pltpu.VMEM((GS * KC + 16,), jnp.int32),
        pltpu.VMEM((GS * KC + 16,), jnp.int32),
        pltpu.VMEM((KC, D), jnp.float32),
        pltpu.VMEM((KC, D), jnp.float32),
        pltpu.SemaphoreType.DMA,
        pltpu.SemaphoreType.DMA,
        pltpu.SemaphoreType.DMA,
        pltpu.SemaphoreType.DMA,
        pltpu.VMEM_SHARED((ACC_ROWS, D), jnp.float32),
    ],
    compiler_params=pltpu.CompilerParams(
        use_tc_tiling_on_sc=False, needs_layout_passes=False),
)
def _agg_kernel(x, edges3, zrows, out, src2, dst2, csrc, clidx,
                rowsA, rowsB, gA, gB, tA, tB, acc):
    c = lax.axis_index("c")
    s = lax.axis_index("s")
    lo = c * HALF

    pltpu.sync_copy(zrows, acc.at[pl.ds(s * ROWS_PER_TILE, ROWS_PER_TILE)])
    plsc.subcore_barrier()

    def lidx_row(j):
        for i in range(KC // 16):
            d = dst2[j, pl.ds(i * 16, 16)]
            inb = (d >= lo) & (d < lo + HALF)
            dst2[j, pl.ds(i * 16, 16)] = jnp.where(
                inb, d - lo, HALF + (d & TRASH_MASK))

    def start_gather(j, buf, sem):
        pltpu.async_copy(x.at[csrc.at[pl.ds(j * KC, KC)]], buf, sem)

    def wait_gather(buf, sem):
        pltpu.make_async_copy(x.at[csrc.at[pl.ds(0, KC)]], buf, sem).wait()

    def start_scatter(j, buf, sem):
        pltpu.async_copy(buf, acc.at[clidx.at[pl.ds(j * KC, KC)]], sem,
                         add=True)

    def wait_scatter(buf, sem):
        pltpu.make_async_copy(buf, acc.at[clidx.at[pl.ds(0, KC)]], sem).wait()

    def super_body(base, nmicro):
        pltpu.sync_copy(edges3.at[0, pl.ds(base, nmicro)],
                        src2.at[pl.ds(0, nmicro)])
        pltpu.sync_copy(edges3.at[1, pl.ds(base, nmicro)],
                        dst2.at[pl.ds(0, nmicro)])

        def comp(v, off):
            r = v >> 3
            cb = (v & 7) * 16
            d = dst2[r, pl.ds(cb, 16)]
            sv = src2[r, pl.ds(cb, 16)]
            inb = (d >= lo) & (d < lo + HALF)
            mi = inb.astype(jnp.int32)
            cum = plsc.cumsum(mi)
            pos = jnp.where(inb, off + cum - mi,
                            GS * KC + lax.iota(jnp.int32, 16))
            plsc.store_scatter(clidx, [pos], d - lo)
            plsc.store_scatter(csrc, [pos], sv)
            pc = plsc.all_reduce_population_count(inb)
            return off + pc[0]
        off = lax.fori_loop(0, nmicro * (KC // 16), comp, 0)

        nmic = (off + KC - 1) >> 7

        def pad(t, carry):
            pos = t * 16 + lax.iota(jnp.int32, 16)
            m = pos >= off
            cl = clidx[pl.ds(t * 16, 16)]
            cs = csrc[pl.ds(t * 16, 16)]
            clidx[pl.ds(t * 16, 16)] = jnp.where(
                m, HALF + (pos & TRASH_MASK), cl)
            csrc[pl.ds(t * 16, 16)] = jnp.where(m, pos, cs)
            return carry
        lax.fori_loop(off >> 4, (nmic * KC) >> 4, pad, 0)

        def fire(m, carry):
            j0 = 2 * m
            j1 = j0 + 1

            @pl.when(m > 0)
            def _():
                wait_scatter(rowsA, tA)
            start_gather(j0, rowsA, gA)

            @pl.when(m > 0)
            def _():
                wait_scatter(rowsB, tB)

            @pl.when(j1 < nmic)
            def _():
                start_gather(j1, rowsB, gB)
            wait_gather(rowsA, gA)
            start_scatter(j0, rowsA, tA)

            @pl.when(j1 < nmic)
            def _():
                wait_gather(rowsB, gB)
                start_scatter(j1, rowsB, tB)
            return carry
        lax.fori_loop(0, (nmic + 1) >> 1, fire, 0)

        @pl.when(nmic > 0)
        def _():
            wait_scatter(rowsA, tA)

        @pl.when((nmic > 0) & ((nmic & 1) == 0))
        def _():
            wait_scatter(rowsB, tB)

    def outer(u, carry):
        super_body(s * CHT + u * GS, GS)
        return carry
    lax.fori_loop(0, SUP, outer, 0)
    if TAILC:
        super_body(s * CHT + SUP * GS, TAILC)

    @pl.when(s < NCHUNK - LEFT)
    def _extra():
        pltpu.sync_copy(edges3.at[0, pl.ds(LEFT + s, 1)], src2.at[pl.ds(0, 1)])
        pltpu.sync_copy(edges3.at[1, pl.ds(LEFT + s, 1)], dst2.at[pl.ds(0, 1)])
        lidx_row(0)
        pltpu.sync_copy(x.at[src2.at[0]], rowsA)
        pltpu.sync_copy(rowsA, acc.at[dst2.at[0]], add=True)

    plsc.subcore_barrier()
    pltpu.sync_copy(acc.at[pl.ds(s * ROWS_PER_TILE, ROWS_PER_TILE)],
                    out.at[c, pl.ds(s * ROWS_PER_TILE, ROWS_PER_TILE)])


BM = 2048


def _mm_body(embT_ref, deg_ref, w_ref, x_ref):
    ns = lax.rsqrt(jnp.clip(deg_ref[0, :, 0:1], 1.0, None))
    y = jnp.dot(jnp.transpose(embT_ref[...]), w_ref[...],
                preferred_element_type=jnp.float32)
    x_ref[...] = y * ns


def _mm_call(embT, deg, W):
    return pl.pallas_call(
        _mm_body,
        grid=(-(-N // BM),),
        in_specs=[
            pl.BlockSpec((D, BM), lambda i: (0, i)),
            pl.BlockSpec((1, BM, DW), lambda i: (0, i, 0)),
            pl.BlockSpec((D, D), lambda i: (0, 0)),
        ],
        out_specs=pl.BlockSpec((BM, D), lambda i: (i, 0)),
        out_shape=jax.ShapeDtypeStruct((N, D), jnp.float32),
    )(embT, deg, W)


BD = 1000
_HB = HALF // BD


def _fin_body(agg_ref, deg_ref, b_ref, out_ref):
    nd = lax.rsqrt(jnp.clip(deg_ref[0, :, 0:1], 1.0, None))
    out_ref[...] = agg_ref[0] * nd + b_ref[...]


def _fin_call(aggs, deg, b2d):
    return pl.pallas_call(
        _fin_body,
        grid=(N // BD,),
        in_specs=[
            pl.BlockSpec((1, BD, D), lambda i: (i // _HB, i % _HB, 0)),
            pl.BlockSpec((1, BD, DW), lambda i: (1, i, 0)),
            pl.BlockSpec((1, D), lambda i: (0, 0)),
        ],
        out_specs=pl.BlockSpec((BD, D), lambda i: (i, 0)),
        out_shape=jax.ShapeDtypeStruct((N, D), jnp.float32),
    )(aggs, deg, b2d)


def kernel(node_embeddings, W, b, edge_index):
    edges3 = edge_index.astype(jnp.int32).reshape(2, NCHUNK, KC)
    ones16 = jnp.ones((KC, DW), jnp.float32)
    zeros16 = jnp.zeros((NPT, DW), jnp.float32)
    deg = _deg_kernel(edges3, ones16, zeros16)
    x = _mm_call(node_embeddings.T, deg, W)
    zrows = jnp.zeros((ROWS_PER_TILE, D), jnp.float32)
    aggs = _agg_kernel(x, edges3, zrows)
    return _fin_call(aggs, deg, b.reshape(1, D))
